# Initial kernel scaffold; baseline (speedup 1.0000x reference)
import jax
import jax.numpy as jnp
from jax.experimental import pallas as pl


def kernel(word_table, postag_table, word_idx, forms_idx, lemmas_idx, postags_idx):
    raise NotImplementedError("write your pallas kernel here")


# speedup vs baseline: 2.1495x; 2.1495x over previous
import functools

import jax
import jax.numpy as jnp
from jax import lax
from jax.experimental import pallas as pl
from jax.experimental.pallas import tpu as pltpu
from jax.experimental.pallas import tpu_sc as plsc


def _morph_kernel(B, D, AL, NC, NW, BW, CH):
    NCHUNK = BW // CH
    mesh = plsc.VectorSubcoreMesh(core_axis_name="c", subcore_axis_name="s")

    @functools.partial(
        pl.kernel,
        mesh=mesh,
        out_type=jax.ShapeDtypeStruct((B, D), jnp.float32),
        scratch_types=[
            pltpu.VMEM((BW,), jnp.int32),
            pltpu.VMEM((BW * AL,), jnp.int32),
            pltpu.VMEM((BW * AL,), jnp.int32),
            pltpu.VMEM((BW * AL,), jnp.int32),
            pltpu.VMEM((BW, D), jnp.float32),
            pltpu.VMEM((CH * AL, D), jnp.float32),
            pltpu.VMEM((CH, D), jnp.float32),
            pltpu.SemaphoreType.DMA,
        ],
        compiler_params=pltpu.CompilerParams(use_tc_tiling_on_sc=False),
    )
    def k(wt, pt, wih, fih, lih, pih, out, idx_w, idx_f, idx_l, idx_p,
          wrows, gbuf, acc, sem):
        cid = lax.axis_index("c")
        sid = lax.axis_index("s")
        wid = sid * NC + cid
        base = pl.multiple_of(wid * BW, BW)

        pltpu.sync_copy(wih.at[pl.ds(base, BW)], idx_w)
        pltpu.sync_copy(fih.at[pl.ds(base * AL, BW * AL)], idx_f)
        pltpu.sync_copy(lih.at[pl.ds(base * AL, BW * AL)], idx_l)
        pltpu.sync_copy(pih.at[pl.ds(base * AL, BW * AL)], idx_p)
        pltpu.async_copy(wt.at[idx_w], wrows, sem).wait()

        def chunk_body(c, _):
            o = pl.multiple_of(c * CH * AL, CH * AL)
            for t, (idxr, tbl) in enumerate(
                    ((idx_f, wt), (idx_l, wt), (idx_p, pt))):
                pltpu.async_copy(tbl.at[idxr.at[pl.ds(o, CH * AL)]],
                                 gbuf, sem).wait()

                def word_body(i, _, t=t):
                    r0 = i * AL
                    a0 = gbuf[r0, pl.ds(0, 16)]
                    a1 = gbuf[r0, pl.ds(16, 16)]
                    for r in range(1, AL):
                        a0 = a0 + gbuf[r0 + r, pl.ds(0, 16)]
                        a1 = a1 + gbuf[r0 + r, pl.ds(16, 16)]
                    if t == 0:
                        acc[i, pl.ds(0, 16)] = a0
                        acc[i, pl.ds(16, 16)] = a1
                    else:
                        acc[i, pl.ds(0, 16)] = acc[i, pl.ds(0, 16)] + a0
                        acc[i, pl.ds(16, 16)] = acc[i, pl.ds(16, 16)] + a1
                    return 0

                lax.fori_loop(0, CH, word_body, 0)

            mscale = jnp.float32(0.25 / AL)
            wscale = jnp.float32(0.25)

            def fin_body(i, _):
                w = c * CH + i
                acc[i, pl.ds(0, 16)] = (acc[i, pl.ds(0, 16)] * mscale
                                        + wrows[w, pl.ds(0, 16)] * wscale)
                acc[i, pl.ds(16, 16)] = (acc[i, pl.ds(16, 16)] * mscale
                                         + wrows[w, pl.ds(16, 16)] * wscale)
                return 0

            lax.fori_loop(0, CH, fin_body, 0)
            pltpu.sync_copy(acc, out.at[pl.ds(base + c * CH, CH)])
            return 0

        lax.fori_loop(0, NCHUNK, chunk_body, 0)

    return k


def kernel(word_table, postag_table, word_idx, forms_idx, lemmas_idx,
           postags_idx):
    B = word_idx.shape[0]
    D = word_table.shape[1]
    AL = forms_idx.shape[1] * forms_idx.shape[2]
    info = plsc.get_sparse_core_info()
    NC, NS = info.num_cores, info.num_subcores
    NW = NC * NS
    BW = B // NW
    CH = 64

    wi = word_idx.astype(jnp.int32)
    fi = forms_idx.reshape(-1).astype(jnp.int32)
    li = lemmas_idx.reshape(-1).astype(jnp.int32)
    pi = postags_idx.reshape(-1).astype(jnp.int32)

    k = _morph_kernel(B, D, AL, NC, NW, BW, CH)
    return k(word_table, postag_table, wi, fi, li, pi)


# speedup vs baseline: 2.7004x; 1.2563x over previous
import functools

import jax
import jax.numpy as jnp
from jax import lax
from jax.experimental import pallas as pl
from jax.experimental.pallas import tpu as pltpu
from jax.experimental.pallas import tpu_sc as plsc


def _morph_kernel(B, D, AL, NC, NW, BW, CH, P1):
    NCHUNK = BW // CH
    NSTEP = NCHUNK * 3
    mesh = plsc.VectorSubcoreMesh(core_axis_name="c", subcore_axis_name="s")

    @functools.partial(
        pl.kernel,
        mesh=mesh,
        out_type=jax.ShapeDtypeStruct((B, D), jnp.float32),
        scratch_types=[
            pltpu.VMEM((BW,), jnp.int32),
            pltpu.VMEM((BW * AL,), jnp.int32),
            pltpu.VMEM((BW * AL,), jnp.int32),
            pltpu.VMEM((BW * AL,), jnp.int32),
            pltpu.VMEM((BW, D), jnp.float32),
            pltpu.VMEM((CH * AL, D), jnp.float32),
            pltpu.VMEM((CH * AL, D), jnp.float32),
            pltpu.VMEM((CH, D), jnp.float32),
            pltpu.VMEM_SHARED((P1, D), jnp.float32),
            pltpu.SemaphoreType.DMA,
            pltpu.SemaphoreType.DMA,
            pltpu.SemaphoreType.DMA,
        ],
        compiler_params=pltpu.CompilerParams(use_tc_tiling_on_sc=False),
    )
    def k(wt, pt, wih, fih, lih, pih, out, idx_w, idx_f, idx_l, idx_p,
          wrows, gbuf0, gbuf1, acc, pts, sem0, sem1, semw):
        cid = lax.axis_index("c")
        sid = lax.axis_index("s")
        wid = sid * NC + cid
        base = pl.multiple_of(wid * BW, BW)

        @pl.when(sid == 0)
        def _():
            pltpu.sync_copy(pt, pts)

        plsc.subcore_barrier()

        pltpu.sync_copy(wih.at[pl.ds(base, BW)], idx_w)
        pltpu.sync_copy(fih.at[pl.ds(base * AL, BW * AL)], idx_f)
        pltpu.sync_copy(lih.at[pl.ds(base * AL, BW * AL)], idx_l)
        pltpu.sync_copy(pih.at[pl.ds(base * AL, BW * AL)], idx_p)
        wdma = pltpu.async_copy(wt.at[idx_w], wrows, semw)

        gb = (gbuf0, gbuf1)
        sems = (sem0, sem1)
        tables = ((idx_f, wt), (idx_l, wt), (idx_p, pts))

        def start(step):
            c, t = divmod(step, 3)
            idxr, tbl = tables[t]
            o = c * CH * AL
            return pltpu.async_copy(tbl.at[idxr.at[pl.ds(o, CH * AL)]],
                                    gb[step % 2], sems[step % 2])

        mscale = jnp.float32(0.25 / AL)
        wscale = jnp.float32(0.25)

        dma = {0: start(0)}
        for step in range(NSTEP):
            c, t = divmod(step, 3)
            if step + 1 < NSTEP:
                dma[step + 1] = start(step + 1)
            dma[step].wait()
            buf = gb[step % 2]

            def word_body(i, _, t=t, buf=buf):
                r0 = i * AL
                a0 = buf[r0, pl.ds(0, 16)]
                a1 = buf[r0, pl.ds(16, 16)]
                for r in range(1, AL):
                    a0 = a0 + buf[r0 + r, pl.ds(0, 16)]
                    a1 = a1 + buf[r0 + r, pl.ds(16, 16)]
                if t == 0:
                    acc[i, pl.ds(0, 16)] = a0
                    acc[i, pl.ds(16, 16)] = a1
                else:
                    acc[i, pl.ds(0, 16)] = acc[i, pl.ds(0, 16)] + a0
                    acc[i, pl.ds(16, 16)] = acc[i, pl.ds(16, 16)] + a1
                return 0

            lax.fori_loop(0, CH, word_body, 0)

            if t == 2:
                if c == 0:
                    wdma.wait()

                def fin_body(i, _, c=c):
                    w = c * CH + i
                    acc[i, pl.ds(0, 16)] = (acc[i, pl.ds(0, 16)] * mscale
                                            + wrows[w, pl.ds(0, 16)] * wscale)
                    acc[i, pl.ds(16, 16)] = (acc[i, pl.ds(16, 16)] * mscale
                                             + wrows[w, pl.ds(16, 16)] * wscale)
                    return 0

                lax.fori_loop(0, CH, fin_body, 0)
                pltpu.sync_copy(acc, out.at[pl.ds(base + c * CH, CH)])

    return k


def kernel(word_table, postag_table, word_idx, forms_idx, lemmas_idx,
           postags_idx):
    B = word_idx.shape[0]
    D = word_table.shape[1]
    AL = forms_idx.shape[1] * forms_idx.shape[2]
    P1 = postag_table.shape[0]
    info = plsc.get_sparse_core_info()
    NC, NS = info.num_cores, info.num_subcores
    NW = NC * NS
    BW = B // NW
    CH = 64

    wi = word_idx.astype(jnp.int32)
    fi = forms_idx.reshape(-1).astype(jnp.int32)
    li = lemmas_idx.reshape(-1).astype(jnp.int32)
    pi = postags_idx.reshape(-1).astype(jnp.int32)

    k = _morph_kernel(B, D, AL, NC, NW, BW, CH, P1)
    return k(word_table, postag_table, wi, fi, li, pi)
